# baseline (speedup 1.0000x reference)
import functools

import jax
import jax.numpy as jnp
from jax import lax
from jax.experimental import pallas as pl
from jax.experimental.pallas import tpu as pltpu
from jax.experimental.pallas import tpu_sc as plsc

NC, NS, LN = 2, 16, 16
B = 128
KBIG = 32
NBUF = 4
NBLK_TC = 2000


def _bcast_lane(v, l):
    idx = jnp.full((LN,), l, dtype=jnp.int32)
    dn = lax.GatherDimensionNumbers(
        offset_dims=(), collapsed_slice_dims=(0,), start_index_map=(0,))
    return lax.gather(v, idx[:, None], dn, (1,),
                      mode=lax.GatherScatterMode.PROMISE_IN_BOUNDS)


def _scale_chunk(gbuf, wbuf, j, dh):
    for g in range(B // LN):
        wv = wbuf[j, pl.ds(g * LN, LN)]
        for l in range(LN):
            bb = _bcast_lane(wv, l)
            e = g * LN + l
            for f in range(dh // LN):
                gbuf[e, pl.ds(f * LN, LN)] = gbuf[e, pl.ds(f * LN, LN)] * bb


def _mesh():
    return plsc.VectorSubcoreMesh(core_axis_name="c", subcore_axis_name="s",
                                  num_cores=NC, num_subcores=NS)


RP, RLAST = 3120, 3200


def _zero_rows(z_h, acc, s):
    row0 = pl.multiple_of(RP * s, 8)

    @pl.when(s < NS - 1)
    def _():
        pltpu.sync_copy(z_h.at[pl.ds(0, RP)], acc.at[pl.ds(row0, RP)])

    @pl.when(s == NS - 1)
    def _():
        pltpu.sync_copy(z_h, acc.at[pl.ds(RP * (NS - 1), RLAST)])


def _write_rows(acc, out, s):
    row0 = pl.multiple_of(RP * s, 8)

    @pl.when(s < NS - 1)
    def _():
        pltpu.sync_copy(acc.at[pl.ds(row0, RP)], out.at[pl.ds(row0, RP)])

    @pl.when(s == NS - 1)
    def _():
        pltpu.sync_copy(acc.at[pl.ds(RP * (NS - 1), RLAST)],
                        out.at[pl.ds(RP * (NS - 1), RLAST)])


def _edge_pipeline(tab, rows_h, cols_h, w_h, acc, rowbuf, colbuf, wbuf,
                   gs, sgs, sss, mybase, nmy, nblk, dh):
    def startg(j, q):
        pltpu.async_copy(tab.at[rowbuf.at[j]], gs[q], sgs[q])

    def waitg(j, q):
        pltpu.make_async_copy(tab.at[rowbuf.at[j]], gs[q], sgs[q]).wait()

    def starts(j, q):
        pltpu.async_copy(gs[q], acc.at[colbuf.at[j]], sss[q], add=True)

    def waits(j, q):
        pltpu.make_async_copy(gs[q], acc.at[colbuf.at[j]], sss[q]).wait()

    def block(bi, _):
        bbase = mybase + bi * KBIG
        cnt = jnp.maximum(0, jnp.minimum(KBIG, nmy - bi * KBIG))

        @pl.when(cnt > 0)
        def _():
            pltpu.sync_copy(rows_h.at[pl.ds(bbase, KBIG)], rowbuf)
            pltpu.sync_copy(cols_h.at[pl.ds(bbase, KBIG)], colbuf)
            pltpu.sync_copy(w_h.at[pl.ds(bbase, KBIG)], wbuf)
            for q in range(NBUF - 1):
                @pl.when(q < cnt)
                def _():
                    startg(q, q)

            def group(g, _):
                j0 = NBUF * g
                for q in range(NBUF):
                    j = j0 + q
                    qn = (q + NBUF - 1) % NBUF

                    @pl.when(j < cnt)
                    def _():
                        waitg(j, q)
                        _scale_chunk(gs[q], wbuf, j, dh)
                        starts(j, q)

                        @pl.when(j + NBUF - 1 < cnt)
                        def _():
                            @pl.when(j >= 1)
                            def _():
                                waits(j - 1, qn)
                            startg(j + NBUF - 1, qn)
                return 0
            lax.fori_loop(0, (cnt + NBUF - 1) // NBUF, group, 0)
            for q in range(NBUF):
                @pl.when(cnt > q)
                def _():
                    waits(0, q)
        return 0
    lax.fori_loop(0, nblk, block, 0)


def _edge_base(c, s):
    return pl.multiple_of(3128 * c + 192 * s, 8)


def _edge_count(c, s):
    last = jnp.where(c == 0, 248, 242)
    return jnp.where(s == NS - 1, last, 192)


def _deg_call(cols2, w2, zdeg):
    n = zdeg.shape[0]

    @functools.partial(
        pl.kernel, mesh=_mesh(),
        compiler_params=pltpu.CompilerParams(use_tc_tiling_on_sc=False),
        out_type=jax.ShapeDtypeStruct((NC, n), jnp.float32),
        scratch_types=[
            pltpu.VMEM((KBIG, B), jnp.int32),
            pltpu.VMEM((KBIG, B), jnp.float32),
            pltpu.VMEM_SHARED((n,), jnp.float32),
        ])
    def k(cols_h, w_h, z_h, out_h, colbuf, wbuf, acc):
        c = lax.axis_index("c")
        s = lax.axis_index("s")

        @pl.when(s == 0)
        def _():
            pltpu.sync_copy(z_h, acc)
        plsc.subcore_barrier()

        nmy = _edge_count(c, s)
        mybase = _edge_base(c, s)

        def block(bi, _):
            bbase = mybase + bi * KBIG
            cnt = jnp.maximum(0, jnp.minimum(KBIG, nmy - bi * KBIG))

            @pl.when(cnt > 0)
            def _():
                pltpu.sync_copy(cols_h.at[pl.ds(bbase, KBIG)], colbuf)
                pltpu.sync_copy(w_h.at[pl.ds(bbase, KBIG)], wbuf)

                def ch(j, _):
                    pltpu.sync_copy(wbuf.at[j], acc.at[colbuf.at[j]],
                                    add=True)
                    return 0
                lax.fori_loop(0, cnt, ch, 0)
            return 0
        lax.fori_loop(0, 8, block, 0)
        plsc.subcore_barrier()

        @pl.when(s == 0)
        def _():
            pltpu.sync_copy(acc, out_h.at[c])

    return k(cols2, w2, zdeg)


def _s1_call(y1, rows2, cols2, w2, z16):
    n = y1.shape[0]
    dh = y1.shape[1]

    @functools.partial(
        pl.kernel, mesh=_mesh(),
        compiler_params=pltpu.CompilerParams(use_tc_tiling_on_sc=False),
        out_type=jax.ShapeDtypeStruct((NC, n, dh), jnp.float32),
        scratch_types=[
            pltpu.VMEM((KBIG, B), jnp.int32),
            pltpu.VMEM((KBIG, B), jnp.int32),
            pltpu.VMEM((KBIG, B), jnp.float32),
            [pltpu.VMEM((B, dh), jnp.float32)] * NBUF,
            pltpu.VMEM_SHARED((n, dh), jnp.float32),
            [pltpu.SemaphoreType.DMA] * NBUF,
            [pltpu.SemaphoreType.DMA] * NBUF,
        ])
    def k(y_h, rows_h, cols_h, w_h, z_h, out_h, rowbuf, colbuf, wbuf,
          gs, acc, sgs, sss):
        c = lax.axis_index("c")
        s = lax.axis_index("s")
        _zero_rows(z_h, acc, s)
        plsc.subcore_barrier()

        nmy = _edge_count(c, s)
        mybase = _edge_base(c, s)
        _edge_pipeline(y_h, rows_h, cols_h, w_h, acc, rowbuf, colbuf, wbuf,
                       gs, sgs, sss, mybase, nmy, 8, dh)
        plsc.subcore_barrier()
        _write_rows(acc, out_h.at[c], s)

    return k(y1, rows2, cols2, w2, z16)


def _s23_call(y2, rows2, cols2, w2, z32):
    n = y2.shape[1]
    dh = y2.shape[2]


    @functools.partial(
        pl.kernel, mesh=_mesh(),
        compiler_params=pltpu.CompilerParams(use_tc_tiling_on_sc=False),
        out_type=jax.ShapeDtypeStruct((NC, n, dh), jnp.float32),
        scratch_types=[
            pltpu.VMEM((KBIG, B), jnp.int32),
            pltpu.VMEM((KBIG, B), jnp.int32),
            pltpu.VMEM((KBIG, B), jnp.float32),
            [pltpu.VMEM((B, dh), jnp.float32)] * NBUF,
            pltpu.VMEM_SHARED((n, dh), jnp.float32),
            [pltpu.SemaphoreType.DMA] * NBUF,
            [pltpu.SemaphoreType.DMA] * NBUF,
        ])
    def k(y_h, rows_h, cols_h, w_h, z_h, out_h, rowbuf, colbuf, wbuf,
          gs, acc, sgs, sss):
        c = lax.axis_index("c")
        s = lax.axis_index("s")
        _zero_rows(z_h, acc, s)
        plsc.subcore_barrier()

        tab = y_h.at[c]
        nmy = jnp.where(s == NS - 1, 490, 384)
        mybase = pl.multiple_of(384 * s, 8)
        _edge_pipeline(tab, rows_h, cols_h, w_h, acc, rowbuf, colbuf, wbuf,
                       gs, sgs, sss, mybase, nmy, 16, dh)
        plsc.subcore_barrier()
        _write_rows(acc, out_h.at[c], s)

    return k(y2, rows2, cols2, w2, z32)


def _tc1(aggp, y1, dis2, w1p, b1):
    n = y1.shape[0]

    def body(aggp_ref, y_ref, dis_ref, w_ref, b_ref, out_ref):
        dis = dis_ref[...][:, :1]
        agg = aggp_ref[0] + aggp_ref[1] + y_ref[...]
        u = dis * agg
        z = jnp.dot(u, w_ref[...], preferred_element_type=jnp.float32)
        h = jax.nn.sigmoid(z + b_ref[...])
        yp = dis * h
        out_ref[0] = yp[:, :32]
        out_ref[1] = yp[:, 32:]

    return pl.pallas_call(
        body, grid=(n // NBLK_TC,),
        in_specs=[
            pl.BlockSpec((2, NBLK_TC, 16), lambda i: (0, i, 0)),
            pl.BlockSpec((NBLK_TC, 16), lambda i: (i, 0)),
            pl.BlockSpec((NBLK_TC, 8), lambda i: (i, 0)),
            pl.BlockSpec((16, 64), lambda i: (0, 0)),
            pl.BlockSpec((1, 64), lambda i: (0, 0)),
        ],
        out_specs=pl.BlockSpec((2, NBLK_TC, 32), lambda i: (0, i, 0)),
        out_shape=jax.ShapeDtypeStruct((2, n, 32), jnp.float32),
    )(aggp, y1, dis2, w1p, b1)


def _tc23(agg, y, dis2, w, b):
    n = y.shape[1]

    def body(agg_ref, y_ref, dis_ref, w_ref, b_ref, out_ref):
        dis = dis_ref[...][:, :1]
        u0 = dis * (agg_ref[0] + y_ref[0])
        u1 = dis * (agg_ref[1] + y_ref[1])
        u = jnp.concatenate([u0, u1], axis=1)
        z = jnp.dot(u, w_ref[...], preferred_element_type=jnp.float32)
        h = jax.nn.sigmoid(z + b_ref[...])
        yp = dis * h
        out_ref[0] = yp[:, :32]
        out_ref[1] = yp[:, 32:]

    return pl.pallas_call(
        body, grid=(n // NBLK_TC,),
        in_specs=[
            pl.BlockSpec((2, NBLK_TC, 32), lambda i: (0, i, 0)),
            pl.BlockSpec((2, NBLK_TC, 32), lambda i: (0, i, 0)),
            pl.BlockSpec((NBLK_TC, 8), lambda i: (i, 0)),
            pl.BlockSpec((64, 64), lambda i: (0, 0)),
            pl.BlockSpec((1, 64), lambda i: (0, 0)),
        ],
        out_specs=pl.BlockSpec((2, NBLK_TC, 32), lambda i: (0, i, 0)),
        out_shape=jax.ShapeDtypeStruct((2, n, 32), jnp.float32),
    )(agg, y, dis2, w, b)


def _tc3(agg, y, dis2, w, b, wl, bl):
    n = y.shape[1]

    def body(agg_ref, y_ref, dis_ref, w_ref, b_ref, wl_ref, bl_ref, out_ref):
        dis = dis_ref[...][:, :1]
        u0 = dis * (agg_ref[0] + y_ref[0])
        u1 = dis * (agg_ref[1] + y_ref[1])
        u = jnp.concatenate([u0, u1], axis=1)
        z = jnp.dot(u, w_ref[...], preferred_element_type=jnp.float32)
        h = jax.nn.sigmoid(z + b_ref[...])
        out_ref[...] = (jnp.dot(h, wl_ref[...],
                                preferred_element_type=jnp.float32)
                        + bl_ref[...])

    return pl.pallas_call(
        body, grid=(n // NBLK_TC,),
        in_specs=[
            pl.BlockSpec((2, NBLK_TC, 32), lambda i: (0, i, 0)),
            pl.BlockSpec((2, NBLK_TC, 32), lambda i: (0, i, 0)),
            pl.BlockSpec((NBLK_TC, 8), lambda i: (i, 0)),
            pl.BlockSpec((64, 64), lambda i: (0, 0)),
            pl.BlockSpec((1, 64), lambda i: (0, 0)),
            pl.BlockSpec((64, 1), lambda i: (0, 0)),
            pl.BlockSpec((1, 1), lambda i: (0, 0)),
        ],
        out_specs=pl.BlockSpec((NBLK_TC, 1), lambda i: (i, 0)),
        out_shape=jax.ShapeDtypeStruct((n, 1), jnp.float32),
    )(agg, y, dis2, w, b, wl, bl)


def _softmax(zr):
    def body(z_ref, o_ref):
        z = z_ref[...]
        m = jnp.max(z)
        e = jnp.exp(z - m)
        o_ref[...] = e / jnp.sum(e)

    return pl.pallas_call(
        body, out_shape=jax.ShapeDtypeStruct(zr.shape, jnp.float32))(zr)


def kernel(vertex_features, edges, weights, W1, b1, W2, b2, W3, b3, Wl, bl):
    x = vertex_features
    n = x.shape[0]
    row = edges[0]
    col = edges[1]

    nch = row.shape[0] // B
    pad_i = jnp.zeros((KBIG, B), jnp.int32)
    pad_f = jnp.zeros((KBIG, B), jnp.float32)
    rows2 = jnp.concatenate([row.reshape(nch, B), pad_i], axis=0)
    cols2 = jnp.concatenate([col.reshape(nch, B), pad_i], axis=0)
    w2e = jnp.concatenate([weights.reshape(nch, B), pad_f], axis=0)

    degp = _deg_call(cols2, w2e, jnp.zeros((n,), jnp.float32))
    deg = degp[0] + degp[1] + 1.0
    dis = lax.rsqrt(deg)
    dis2 = jnp.tile(dis[:, None], (1, 8))

    x16 = jnp.pad(x, ((0, 0), (0, 16 - x.shape[1])))
    y1 = dis[:, None] * x16
    z16 = jnp.zeros((RLAST, 16), jnp.float32)
    z32 = jnp.zeros((RLAST, 32), jnp.float32)

    aggp1 = _s1_call(y1, rows2, cols2, w2e, z16)
    w1p = jnp.pad(W1, ((0, 16 - W1.shape[0]), (0, 0)))
    y2 = _tc1(aggp1, y1, dis2, w1p, b1.reshape(1, -1))

    agg2 = _s23_call(y2, rows2, cols2, w2e, z32)
    y3 = _tc23(agg2, y2, dis2, W2, b2.reshape(1, -1))

    agg3 = _s23_call(y3, rows2, cols2, w2e, z32)
    z = _tc3(agg3, y3, dis2, W3, b3.reshape(1, -1),
             Wl, bl.reshape(1, 1))

    out = _softmax(z.reshape(400, 125)).reshape(n, 1)
    return out


# speedup vs baseline: 1.0141x; 1.0141x over previous
import functools

import jax
import jax.numpy as jnp
from jax import lax
from jax.experimental import pallas as pl
from jax.experimental.pallas import tpu as pltpu
from jax.experimental.pallas import tpu_sc as plsc

NC, NS, LN = 2, 16, 16
B = 128
KBIG = 32
NBUF = 4
NBLK_TC = 2000


def _bcast_lane(v, l):
    idx = jnp.full((LN,), l, dtype=jnp.int32)
    dn = lax.GatherDimensionNumbers(
        offset_dims=(), collapsed_slice_dims=(0,), start_index_map=(0,))
    return lax.gather(v, idx[:, None], dn, (1,),
                      mode=lax.GatherScatterMode.PROMISE_IN_BOUNDS)


def _scale_chunk(gbuf, wbuf, j, dh):
    for g in range(B // LN):
        wv = wbuf[j, pl.ds(g * LN, LN)]
        for l in range(LN):
            bb = _bcast_lane(wv, l)
            e = g * LN + l
            for f in range(dh // LN):
                gbuf[e, pl.ds(f * LN, LN)] = gbuf[e, pl.ds(f * LN, LN)] * bb


def _mesh():
    return plsc.VectorSubcoreMesh(core_axis_name="c", subcore_axis_name="s",
                                  num_cores=NC, num_subcores=NS)


RP, RLAST = 3120, 3200


def _zero_rows(z_h, acc, s):
    row0 = pl.multiple_of(RP * s, 8)

    @pl.when(s < NS - 1)
    def _():
        pltpu.sync_copy(z_h.at[pl.ds(0, RP)], acc.at[pl.ds(row0, RP)])

    @pl.when(s == NS - 1)
    def _():
        pltpu.sync_copy(z_h, acc.at[pl.ds(RP * (NS - 1), RLAST)])


def _write_rows(acc, out, s):
    row0 = pl.multiple_of(RP * s, 8)

    @pl.when(s < NS - 1)
    def _():
        pltpu.sync_copy(acc.at[pl.ds(row0, RP)], out.at[pl.ds(row0, RP)])

    @pl.when(s == NS - 1)
    def _():
        pltpu.sync_copy(acc.at[pl.ds(RP * (NS - 1), RLAST)],
                        out.at[pl.ds(RP * (NS - 1), RLAST)])


def _edge_pipeline(tab, rows_h, cols_h, w_h, acc, rowbuf, colbuf, wbuf,
                   gs, sgs, sss, mybase, nmy, nblk, dh):
    def startg(j, q):
        pltpu.async_copy(tab.at[rowbuf.at[j]], gs[q], sgs[q])

    def waitg(j, q):
        pltpu.make_async_copy(tab.at[rowbuf.at[j]], gs[q], sgs[q]).wait()

    def starts(j, q):
        pltpu.async_copy(gs[q], acc.at[colbuf.at[j]], sss[q], add=True)

    def waits(j, q):
        pltpu.make_async_copy(gs[q], acc.at[colbuf.at[j]], sss[q]).wait()

    def block(bi, _):
        bbase = mybase + bi * KBIG
        cnt = jnp.maximum(0, jnp.minimum(KBIG, nmy - bi * KBIG))

        @pl.when(cnt > 0)
        def _():
            pltpu.sync_copy(rows_h.at[pl.ds(bbase, KBIG)], rowbuf)
            pltpu.sync_copy(cols_h.at[pl.ds(bbase, KBIG)], colbuf)
            pltpu.sync_copy(w_h.at[pl.ds(bbase, KBIG)], wbuf)
            for q in range(NBUF - 1):
                @pl.when(q < cnt)
                def _():
                    startg(q, q)

            def group(g, _):
                j0 = NBUF * g
                for q in range(NBUF):
                    j = j0 + q
                    qn = (q + NBUF - 1) % NBUF

                    @pl.when(j < cnt)
                    def _():
                        waitg(j, q)
                        _scale_chunk(gs[q], wbuf, j, dh)
                        starts(j, q)

                        @pl.when(j + NBUF - 1 < cnt)
                        def _():
                            @pl.when(j >= 1)
                            def _():
                                waits(j - 1, qn)
                            startg(j + NBUF - 1, qn)
                return 0
            lax.fori_loop(0, (cnt + NBUF - 1) // NBUF, group, 0)
            for q in range(NBUF):
                @pl.when(cnt > q)
                def _():
                    waits(0, q)
        return 0
    lax.fori_loop(0, nblk, block, 0)


def _edge_base(c, s):
    return pl.multiple_of(3128 * c + 192 * s, 8)


def _edge_count(c, s):
    last = jnp.where(c == 0, 248, 242)
    return jnp.where(s == NS - 1, last, 192)


def _deg_call(cols2, w2, zdeg):
    n = zdeg.shape[0]

    @functools.partial(
        pl.kernel, mesh=_mesh(),
        compiler_params=pltpu.CompilerParams(use_tc_tiling_on_sc=False),
        out_type=jax.ShapeDtypeStruct((NC, n), jnp.float32),
        scratch_types=[
            pltpu.VMEM((KBIG, B), jnp.int32),
            pltpu.VMEM((KBIG, B), jnp.float32),
            pltpu.VMEM_SHARED((n,), jnp.float32),
        ])
    def k(cols_h, w_h, z_h, out_h, colbuf, wbuf, acc):
        c = lax.axis_index("c")
        s = lax.axis_index("s")

        @pl.when(s == 0)
        def _():
            pltpu.sync_copy(z_h, acc)
        plsc.subcore_barrier()

        nmy = _edge_count(c, s)
        mybase = _edge_base(c, s)

        def block(bi, _):
            bbase = mybase + bi * KBIG
            cnt = jnp.maximum(0, jnp.minimum(KBIG, nmy - bi * KBIG))

            @pl.when(cnt > 0)
            def _():
                pltpu.sync_copy(cols_h.at[pl.ds(bbase, KBIG)], colbuf)
                pltpu.sync_copy(w_h.at[pl.ds(bbase, KBIG)], wbuf)

                def ch(j, _):
                    pltpu.sync_copy(wbuf.at[j], acc.at[colbuf.at[j]],
                                    add=True)
                    return 0
                lax.fori_loop(0, cnt, ch, 0)
            return 0
        lax.fori_loop(0, 8, block, 0)
        plsc.subcore_barrier()

        @pl.when(s == 0)
        def _():
            pltpu.sync_copy(acc, out_h.at[c])

    return k(cols2, w2, zdeg)


def _s1_call(y1, rows2, cols2, w2, z16):
    n = y1.shape[0]
    dh = y1.shape[1]

    @functools.partial(
        pl.kernel, mesh=_mesh(),
        compiler_params=pltpu.CompilerParams(use_tc_tiling_on_sc=False),
        out_type=jax.ShapeDtypeStruct((NC, n, dh), jnp.float32),
        scratch_types=[
            pltpu.VMEM((KBIG, B), jnp.int32),
            pltpu.VMEM((KBIG, B), jnp.int32),
            pltpu.VMEM((KBIG, B), jnp.float32),
            [pltpu.VMEM((B, dh), jnp.float32)] * NBUF,
            pltpu.VMEM_SHARED((n, dh), jnp.float32),
            [pltpu.SemaphoreType.DMA] * NBUF,
            [pltpu.SemaphoreType.DMA] * NBUF,
        ])
    def k(y_h, rows_h, cols_h, w_h, z_h, out_h, rowbuf, colbuf, wbuf,
          gs, acc, sgs, sss):
        c = lax.axis_index("c")
        s = lax.axis_index("s")
        _zero_rows(z_h, acc, s)
        plsc.subcore_barrier()

        nmy = _edge_count(c, s)
        mybase = _edge_base(c, s)
        _edge_pipeline(y_h, rows_h, cols_h, w_h, acc, rowbuf, colbuf, wbuf,
                       gs, sgs, sss, mybase, nmy, 8, dh)
        plsc.subcore_barrier()
        _write_rows(acc, out_h.at[c], s)

    return k(y1, rows2, cols2, w2, z16)


def _s23_call(y2, rows2, cols2, w2, z32):
    n = y2.shape[1]
    dh = y2.shape[2]


    @functools.partial(
        pl.kernel, mesh=_mesh(),
        compiler_params=pltpu.CompilerParams(use_tc_tiling_on_sc=False),
        out_type=jax.ShapeDtypeStruct((NC, n, dh), jnp.float32),
        scratch_types=[
            pltpu.VMEM((KBIG, B), jnp.int32),
            pltpu.VMEM((KBIG, B), jnp.int32),
            pltpu.VMEM((KBIG, B), jnp.float32),
            [pltpu.VMEM((B, dh), jnp.float32)] * NBUF,
            pltpu.VMEM_SHARED((n, dh), jnp.float32),
            [pltpu.SemaphoreType.DMA] * NBUF,
            [pltpu.SemaphoreType.DMA] * NBUF,
        ])
    def k(y_h, rows_h, cols_h, w_h, z_h, out_h, rowbuf, colbuf, wbuf,
          gs, acc, sgs, sss):
        c = lax.axis_index("c")
        s = lax.axis_index("s")
        _zero_rows(z_h, acc, s)
        plsc.subcore_barrier()

        tab = y_h.at[c]
        nmy = jnp.where(s == NS - 1, 490, 384)
        mybase = pl.multiple_of(384 * s, 8)
        _edge_pipeline(tab, rows_h, cols_h, w_h, acc, rowbuf, colbuf, wbuf,
                       gs, sgs, sss, mybase, nmy, 16, dh)
        plsc.subcore_barrier()
        _write_rows(acc, out_h.at[c], s)

    return k(y2, rows2, cols2, w2, z32)


def _tc1(aggp, y1, dis2, w1p, b1):
    n = y1.shape[0]

    def body(aggp_ref, y_ref, dis_ref, w_ref, b_ref, out_ref):
        dis = dis_ref[...][:, :1]
        agg = aggp_ref[0] + aggp_ref[1] + y_ref[...]
        u = dis * agg
        z = jnp.dot(u, w_ref[...], preferred_element_type=jnp.float32)
        h = jax.nn.sigmoid(z + b_ref[...])
        yp = dis * h
        out_ref[0] = yp[:, :32]
        out_ref[1] = yp[:, 32:]

    return pl.pallas_call(
        body, grid=(n // NBLK_TC,),
        in_specs=[
            pl.BlockSpec((2, NBLK_TC, 16), lambda i: (0, i, 0)),
            pl.BlockSpec((NBLK_TC, 16), lambda i: (i, 0)),
            pl.BlockSpec((NBLK_TC, 1), lambda i: (i, 0)),
            pl.BlockSpec((16, 64), lambda i: (0, 0)),
            pl.BlockSpec((1, 64), lambda i: (0, 0)),
        ],
        out_specs=pl.BlockSpec((2, NBLK_TC, 32), lambda i: (0, i, 0)),
        out_shape=jax.ShapeDtypeStruct((2, n, 32), jnp.float32),
    )(aggp, y1, dis2, w1p, b1)


def _tc23(agg, y, dis2, w, b):
    n = y.shape[1]

    def body(agg_ref, y_ref, dis_ref, w_ref, b_ref, out_ref):
        dis = dis_ref[...][:, :1]
        u0 = dis * (agg_ref[0] + y_ref[0])
        u1 = dis * (agg_ref[1] + y_ref[1])
        u = jnp.concatenate([u0, u1], axis=1)
        z = jnp.dot(u, w_ref[...], preferred_element_type=jnp.float32)
        h = jax.nn.sigmoid(z + b_ref[...])
        yp = dis * h
        out_ref[0] = yp[:, :32]
        out_ref[1] = yp[:, 32:]

    return pl.pallas_call(
        body, grid=(n // NBLK_TC,),
        in_specs=[
            pl.BlockSpec((2, NBLK_TC, 32), lambda i: (0, i, 0)),
            pl.BlockSpec((2, NBLK_TC, 32), lambda i: (0, i, 0)),
            pl.BlockSpec((NBLK_TC, 1), lambda i: (i, 0)),
            pl.BlockSpec((64, 64), lambda i: (0, 0)),
            pl.BlockSpec((1, 64), lambda i: (0, 0)),
        ],
        out_specs=pl.BlockSpec((2, NBLK_TC, 32), lambda i: (0, i, 0)),
        out_shape=jax.ShapeDtypeStruct((2, n, 32), jnp.float32),
    )(agg, y, dis2, w, b)


def _tc3(agg, y, dis2, w, b, wl, bl):
    n = y.shape[1]

    def body(agg_ref, y_ref, dis_ref, w_ref, b_ref, wl_ref, bl_ref, out_ref):
        dis = dis_ref[...][:, :1]
        u0 = dis * (agg_ref[0] + y_ref[0])
        u1 = dis * (agg_ref[1] + y_ref[1])
        u = jnp.concatenate([u0, u1], axis=1)
        z = jnp.dot(u, w_ref[...], preferred_element_type=jnp.float32)
        h = jax.nn.sigmoid(z + b_ref[...])
        out_ref[...] = (jnp.dot(h, wl_ref[...],
                                preferred_element_type=jnp.float32)
                        + bl_ref[...])

    return pl.pallas_call(
        body, grid=(n // NBLK_TC,),
        in_specs=[
            pl.BlockSpec((2, NBLK_TC, 32), lambda i: (0, i, 0)),
            pl.BlockSpec((2, NBLK_TC, 32), lambda i: (0, i, 0)),
            pl.BlockSpec((NBLK_TC, 1), lambda i: (i, 0)),
            pl.BlockSpec((64, 64), lambda i: (0, 0)),
            pl.BlockSpec((1, 64), lambda i: (0, 0)),
            pl.BlockSpec((64, 1), lambda i: (0, 0)),
            pl.BlockSpec((1, 1), lambda i: (0, 0)),
        ],
        out_specs=pl.BlockSpec((NBLK_TC, 1), lambda i: (i, 0)),
        out_shape=jax.ShapeDtypeStruct((n, 1), jnp.float32),
    )(agg, y, dis2, w, b, wl, bl)


def _softmax(zr):
    def body(z_ref, o_ref):
        z = z_ref[...]
        m = jnp.max(z)
        e = jnp.exp(z - m)
        o_ref[...] = e / jnp.sum(e)

    return pl.pallas_call(
        body, out_shape=jax.ShapeDtypeStruct(zr.shape, jnp.float32))(zr)


def kernel(vertex_features, edges, weights, W1, b1, W2, b2, W3, b3, Wl, bl):
    x = vertex_features
    n = x.shape[0]
    row = edges[0]
    col = edges[1]

    nch = row.shape[0] // B
    pad_i = jnp.zeros((KBIG, B), jnp.int32)
    pad_f = jnp.zeros((KBIG, B), jnp.float32)
    rows2 = jnp.concatenate([row.reshape(nch, B), pad_i], axis=0)
    cols2 = jnp.concatenate([col.reshape(nch, B), pad_i], axis=0)
    w2e = jnp.concatenate([weights.reshape(nch, B), pad_f], axis=0)

    degp = _deg_call(cols2, w2e, jnp.zeros((n,), jnp.float32))
    deg = degp[0] + degp[1] + 1.0
    dis = lax.rsqrt(deg)
    dis2 = dis[:, None]

    x16 = jnp.pad(x, ((0, 0), (0, 16 - x.shape[1])))
    y1 = dis[:, None] * x16
    z16 = jnp.zeros((RLAST, 16), jnp.float32)
    z32 = jnp.zeros((RLAST, 32), jnp.float32)

    aggp1 = _s1_call(y1, rows2, cols2, w2e, z16)
    w1p = jnp.pad(W1, ((0, 16 - W1.shape[0]), (0, 0)))
    y2 = _tc1(aggp1, y1, dis2, w1p, b1.reshape(1, -1))

    agg2 = _s23_call(y2, rows2, cols2, w2e, z32)
    y3 = _tc23(agg2, y2, dis2, W2, b2.reshape(1, -1))

    agg3 = _s23_call(y3, rows2, cols2, w2e, z32)
    z = _tc3(agg3, y3, dis2, W3, b3.reshape(1, -1),
             Wl, bl.reshape(1, 1))

    out = _softmax(z.reshape(400, 125)).reshape(n, 1)
    return out
